# Initial kernel scaffold; baseline (speedup 1.0000x reference)
import jax
import jax.numpy as jnp
from jax.experimental import pallas as pl


def kernel(node_fts, gkt_edge_fts, hidden, cfg_indices_padded, gkt_indices_padded, W_m, b_m, W_skip, b_skip, W_a1, b_a1, W_a2, b_a2, W_ae, b_ae):
    raise NotImplementedError("write your pallas kernel here")


# speedup vs baseline: 11.3508x; 11.3508x over previous
import functools

import jax
import jax.numpy as jnp
from jax.experimental import pallas as pl
from jax.experimental.pallas import tpu as pltpu

N = 10000
D = 128
OUT = 128
H = 8
HS = OUT // H


def _dense_proj_body(nf_ref, hd_ref, wa_ref, ba_ref, wvs_ref, bvs_ref,
                     att_ref, val_ref, skip_ref):
    z = jnp.concatenate([nf_ref[...], hd_ref[...]], axis=-1)
    att_ref[...] = z @ wa_ref[...] + ba_ref[...]
    vs = z @ wvs_ref[...] + bvs_ref[...]
    val_ref[...] = vs[:, :OUT]
    skip_ref[...] = vs[:, OUT:]


def _dense_proj(nf, hd, Wa, ba, Wvs, bvs):
    n = nf.shape[0]
    return pl.pallas_call(
        _dense_proj_body,
        out_shape=[
            jax.ShapeDtypeStruct((n, 32), jnp.float32),
            jax.ShapeDtypeStruct((n, OUT), jnp.float32),
            jax.ShapeDtypeStruct((n, OUT), jnp.float32),
        ],
    )(nf, hd, Wa, ba, Wvs, bvs)


def _epilogue_body(u_ref, den_ref, skip_ref, out_ref):
    out_ref[...] = jnp.maximum(u_ref[...] / den_ref[...] + skip_ref[...], 0.0)


def _epilogue(u, den128, skip):
    n = u.shape[0]
    return pl.pallas_call(
        _epilogue_body,
        out_shape=jax.ShapeDtypeStruct((n, OUT), jnp.float32),
    )(u, den128, skip)


def _sparse_phase(att, values, src, tgt, ae_pad):
    rowS = att[src]
    rowT = att[tgt]
    if ae_pad is None:
        logits = rowS[:, 0:8] + rowT[:, 16:24]
    else:
        logits = rowT[:, 0:8] + rowS[:, 16:24] + ae_pad[:, :8]
    logits = jax.nn.leaky_relu(logits)
    p = jnp.exp(logits)
    den = jax.ops.segment_sum(p, tgt, N)
    vsrc = values[src].reshape(-1, H, HS)
    msg = (p[..., None] * vsrc).reshape(-1, OUT)
    u = jax.ops.segment_sum(msg, tgt, N)
    return u, den


def kernel(node_fts, gkt_edge_fts, hidden, cfg_indices_padded, gkt_indices_padded,
           W_m, b_m, W_skip, b_skip, W_a1, b_a1, W_a2, b_a2, W_ae, b_ae):
    nf = node_fts[0]
    hd = hidden[0]
    cfg_src = cfg_indices_padded[0, :, 0]
    cfg_tgt = cfg_indices_padded[0, :, 1]
    gkt_src = gkt_indices_padded[0, :, 0]
    gkt_tgt = gkt_indices_padded[0, :, 1]

    zpad8 = jnp.zeros((2 * D, 8), jnp.float32)
    Wa = jnp.concatenate([W_a1, zpad8, W_a2, zpad8], axis=1)
    ba = jnp.concatenate([b_a1, jnp.zeros(8), b_a2, jnp.zeros(8)]).astype(jnp.float32)
    Wvs = jnp.concatenate([W_m, W_skip], axis=1)
    bvs = jnp.concatenate([b_m, b_skip]).astype(jnp.float32)

    ae = gkt_edge_fts[0] @ W_ae + b_ae
    ae_pad = jnp.pad(ae, ((0, 0), (0, 8)))

    att, values, skip = _dense_proj(nf, hd, Wa, ba, Wvs, bvs)
    u, den = _sparse_phase(att, values, cfg_src, cfg_tgt, None)
    den128 = jnp.repeat(den, HS, axis=-1)
    cfg_hidden = _epilogue(u, den128, skip)

    att2, values2, skip2 = _dense_proj(nf, cfg_hidden, Wa, ba, Wvs, bvs)
    u2, den2 = _sparse_phase(att2, values2, gkt_src, gkt_tgt, ae_pad)
    den128_2 = jnp.repeat(den2, HS, axis=-1)
    ret = _epilogue(u2, den128_2, skip2)
    return ret[None]


# speedup vs baseline: 37.7782x; 3.3282x over previous
import functools

import jax
import jax.numpy as jnp
from jax import lax
from jax.experimental import pallas as pl
from jax.experimental.pallas import tpu as pltpu
from jax.experimental.pallas import tpu_sc as plsc

N = 10000
D = 128
OUT = 128
H = 8
HS = OUT // H

NC, NS = 2, 16
NT = 10240
RT = NT // NS
E = 160000
EP = 163840
EW = EP // (NC * NS)
K = 128
NCH = EW // K


RB = 2048
EB = 8000


def _proj_body(nf_ref, hd_ref, wa_ref, ba_ref, wvs_ref, bvs_ref,
               att_ref, val_ref, skip_ref):
    z = jnp.concatenate([nf_ref[...], hd_ref[...]], axis=-1)
    att_ref[...] = z @ wa_ref[...] + ba_ref[...]
    vs = z @ wvs_ref[...] + bvs_ref[...]
    val_ref[...] = vs[:, :OUT]
    skip_ref[...] = vs[:, OUT:]


def _row_spec(w):
    return pl.BlockSpec((RB, w), lambda i: (i, 0))


def _full_spec(shape):
    return pl.BlockSpec(shape, lambda i: tuple(0 for _ in shape))


def _proj(nf, hd, Wa, ba, Wvs, bvs):
    return pl.pallas_call(
        _proj_body,
        grid=(NT // RB,),
        in_specs=[
            _row_spec(D), _row_spec(D),
            _full_spec((2 * D, 32)), _full_spec((1, 32)),
            _full_spec((2 * D, 2 * OUT)), _full_spec((1, 2 * OUT)),
        ],
        out_specs=[_row_spec(32), _row_spec(OUT), _row_spec(OUT)],
        out_shape=[
            jax.ShapeDtypeStruct((NT, 32), jnp.float32),
            jax.ShapeDtypeStruct((NT, OUT), jnp.float32),
            jax.ShapeDtypeStruct((NT, OUT), jnp.float32),
        ],
    )(nf, hd, Wa, ba, Wvs, bvs)


def _ae_body(ef_ref, w_ref, b_ref, out_ref):
    out_ref[...] = ef_ref[...] @ w_ref[...] + b_ref[...]


def _ae_proj(ef, Wae, bae):
    return pl.pallas_call(
        _ae_body,
        grid=(E // EB,),
        in_specs=[
            pl.BlockSpec((EB, 16), lambda i: (i, 0)),
            _full_spec((16, 16)), _full_spec((1, 16)),
        ],
        out_specs=pl.BlockSpec((EB, 16), lambda i: (i, 0)),
        out_shape=jax.ShapeDtypeStruct((E, 16), jnp.float32),
    )(ef, Wae, bae)


def _bridge_body(nf_ref, u_ref, den_ref, skip_ref, ex_ref,
                 wa_ref, ba_ref, wvs_ref, bvs_ref,
                 att_ref, val_ref, skip2_ref):
    den128 = (den_ref[0, :, 0:8] + den_ref[1, :, 0:8]) @ ex_ref[...]
    h = jnp.maximum((u_ref[0] + u_ref[1]) / den128 + skip_ref[...], 0.0)
    z = jnp.concatenate([nf_ref[...], h], axis=-1)
    att_ref[...] = z @ wa_ref[...] + ba_ref[...]
    vs = z @ wvs_ref[...] + bvs_ref[...]
    val_ref[...] = vs[:, :OUT]
    skip2_ref[...] = vs[:, OUT:]


def _pair_spec(w):
    return pl.BlockSpec((NC, RB, w), lambda i: (0, i, 0))


def _bridge(nf, u, den, skip, ex, Wa, ba, Wvs, bvs):
    return pl.pallas_call(
        _bridge_body,
        grid=(NT // RB,),
        in_specs=[
            _row_spec(D), _pair_spec(OUT), _pair_spec(16), _row_spec(OUT),
            _full_spec((H, OUT)),
            _full_spec((2 * D, 32)), _full_spec((1, 32)),
            _full_spec((2 * D, 2 * OUT)), _full_spec((1, 2 * OUT)),
        ],
        out_specs=[_row_spec(32), _row_spec(OUT), _row_spec(OUT)],
        out_shape=[
            jax.ShapeDtypeStruct((NT, 32), jnp.float32),
            jax.ShapeDtypeStruct((NT, OUT), jnp.float32),
            jax.ShapeDtypeStruct((NT, OUT), jnp.float32),
        ],
    )(nf, u, den, skip, ex, Wa, ba, Wvs, bvs)


def _final_body(u_ref, den_ref, skip_ref, ex_ref, out_ref):
    den128 = (den_ref[0, :, 0:8] + den_ref[1, :, 0:8]) @ ex_ref[...]
    out_ref[...] = jnp.maximum((u_ref[0] + u_ref[1]) / den128 + skip_ref[...], 0.0)


def _final(u, den, skip, ex):
    return pl.pallas_call(
        _final_body,
        grid=(NT // RB,),
        in_specs=[_pair_spec(OUT), _pair_spec(16), _row_spec(OUT),
                  _full_spec((H, OUT))],
        out_specs=_row_spec(OUT),
        out_shape=jax.ShapeDtypeStruct((NT, OUT), jnp.float32),
    )(u, den, skip, ex)


def _sc_body(swap_roles, has_ae, *refs):
    if has_ae:
        (att_hbm, val_hbm, src_hbm, tgt_hbm, ae_hbm, u_out, den_out,
         u_sh, den_sh,
         idxS, idxT, aS, aT, aeb, pbuf, vrows, zden,
         sem_a, sem_b, sem_v) = refs
    else:
        (att_hbm, val_hbm, src_hbm, tgt_hbm, u_out, den_out,
         u_sh, den_sh,
         idxS, idxT, aS, aT, pbuf, vrows, zden,
         sem_a, sem_b, sem_v) = refs
        ae_hbm = aeb = None

    c = lax.axis_index("c")
    s = lax.axis_index("s")
    wid = c * NS + s
    row0 = s * RT
    zv = jnp.zeros((16,), jnp.float32)

    def zrow(i, _):
        for j in range(8):
            vrows[i, pl.ds(16 * j, 16)] = zv
        zden[i, :] = zv
        return 0
    lax.fori_loop(0, K, zrow, 0)
    for b in range(RT // K):
        pltpu.sync_copy(vrows, u_sh.at[pl.ds(row0 + b * K, K)])
        pltpu.sync_copy(zden, den_sh.at[pl.ds(row0 + b * K, K)])
    plsc.subcore_barrier()

    def chunk(j, _):
        base = wid * EW + j * K
        pltpu.sync_copy(src_hbm.at[pl.ds(base, K)], idxS)
        pltpu.sync_copy(tgt_hbm.at[pl.ds(base, K)], idxT)
        cpv = pltpu.async_copy(val_hbm.at[idxS], vrows, sem_v)
        cpa = pltpu.async_copy(att_hbm.at[idxS], aS, sem_a)
        cpb = pltpu.async_copy(att_hbm.at[idxT], aT, sem_b)
        if has_ae:
            pltpu.sync_copy(ae_hbm.at[pl.ds(base, K)], aeb)
        cpa.wait()
        cpb.wait()

        def pcalc(e, _):
            if swap_roles:
                x = aT[e, pl.ds(0, 16)] + aS[e, pl.ds(16, 16)]
            else:
                x = aS[e, pl.ds(0, 16)] + aT[e, pl.ds(16, 16)]
            if has_ae:
                x = x + aeb[e, :]
            x = jnp.where(x > 0, x, 0.01 * x)
            pbuf[e, :] = jnp.exp(x)
            return 0
        lax.fori_loop(0, K, pcalc, 0)
        pltpu.sync_copy(pbuf, den_sh.at[idxT], add=True)

        cpv.wait()

        def scale(e, _):
            pv = pbuf[e, :]
            for h in range(H):
                w = jnp.broadcast_to(pv[h], (16,))
                vrows[e, pl.ds(HS * h, HS)] = vrows[e, pl.ds(HS * h, HS)] * w
            return 0
        lax.fori_loop(0, K, scale, 0)
        pltpu.sync_copy(vrows, u_sh.at[idxT], add=True)
        return 0

    lax.fori_loop(0, NCH, chunk, 0)
    plsc.subcore_barrier()

    pltpu.sync_copy(u_sh.at[pl.ds(row0, RT)], u_out.at[c, pl.ds(row0, RT)])
    pltpu.sync_copy(den_sh.at[pl.ds(row0, RT)], den_out.at[c, pl.ds(row0, RT)])


def _make_sc_phase(swap_roles, has_ae):
    scratch = [
        pltpu.VMEM_SHARED((NT, OUT), jnp.float32),
        pltpu.VMEM_SHARED((NT, 16), jnp.float32),
        pltpu.VMEM((K,), jnp.int32),
        pltpu.VMEM((K,), jnp.int32),
        pltpu.VMEM((K, 32), jnp.float32),
        pltpu.VMEM((K, 32), jnp.float32),
    ]
    if has_ae:
        scratch.append(pltpu.VMEM((K, 16), jnp.float32))
    scratch += [
        pltpu.VMEM((K, 16), jnp.float32),
        pltpu.VMEM((K, OUT), jnp.float32),
        pltpu.VMEM((K, 16), jnp.float32),
        pltpu.SemaphoreType.DMA,
        pltpu.SemaphoreType.DMA,
        pltpu.SemaphoreType.DMA,
    ]
    return pl.kernel(
        functools.partial(_sc_body, swap_roles, has_ae),
        out_type=[
            jax.ShapeDtypeStruct((NC, NT, OUT), jnp.float32),
            jax.ShapeDtypeStruct((NC, NT, 16), jnp.float32),
        ],
        mesh=plsc.VectorSubcoreMesh(core_axis_name="c", subcore_axis_name="s"),
        scratch_types=scratch,
        compiler_params=pltpu.CompilerParams(use_tc_tiling_on_sc=False),
    )


_sc_phase1 = _make_sc_phase(swap_roles=False, has_ae=False)
_sc_phase2 = _make_sc_phase(swap_roles=True, has_ae=True)


def kernel(node_fts, gkt_edge_fts, hidden, cfg_indices_padded, gkt_indices_padded,
           W_m, b_m, W_skip, b_skip, W_a1, b_a1, W_a2, b_a2, W_ae, b_ae):
    nf = jnp.pad(node_fts[0], ((0, NT - N), (0, 0)))
    hd = jnp.pad(hidden[0], ((0, NT - N), (0, 0)))

    def pad_idx(ix):
        return jnp.pad(ix, (0, EP - E), constant_values=N)

    cfg_src = pad_idx(cfg_indices_padded[0, :, 0])
    cfg_tgt = pad_idx(cfg_indices_padded[0, :, 1])
    gkt_src = pad_idx(gkt_indices_padded[0, :, 0])
    gkt_tgt = pad_idx(gkt_indices_padded[0, :, 1])

    zpad8 = jnp.zeros((2 * D, 8), jnp.float32)
    Wa = jnp.concatenate([W_a1, zpad8, W_a2, zpad8], axis=1)
    ba = jnp.concatenate([b_a1, jnp.zeros(8, jnp.float32),
                          b_a2, jnp.zeros(8, jnp.float32)])[None]
    Wvs = jnp.concatenate([W_m, W_skip], axis=1)
    bvs = jnp.concatenate([b_m, b_skip])[None]
    Wae = jnp.pad(W_ae, ((0, 0), (0, 8)))
    bae = jnp.pad(b_ae, (0, 8))[None]
    ex = jnp.repeat(jnp.eye(H, dtype=jnp.float32), HS, axis=1)

    ae = _ae_proj(gkt_edge_fts[0], Wae, bae)
    ae = jnp.pad(ae, ((0, EP - E), (0, 0)))

    att, values, skip = _proj(nf, hd, Wa, ba, Wvs, bvs)
    u, den = _sc_phase1(att, values, cfg_src, cfg_tgt)

    att2, values2, skip2 = _bridge(nf, u, den, skip, ex, Wa, ba, Wvs, bvs)

    u2, den2 = _sc_phase2(att2, values2, gkt_src, gkt_tgt, ae)
    ret = _final(u2, den2, skip2, ex)
    return ret[:N][None]


# speedup vs baseline: 60.1344x; 1.5918x over previous
import functools

import jax
import jax.numpy as jnp
from jax import lax
from jax.experimental import pallas as pl
from jax.experimental.pallas import tpu as pltpu
from jax.experimental.pallas import tpu_sc as plsc

N = 10000
D = 128
OUT = 128
H = 8
HS = OUT // H

NC, NS = 2, 16
NT = 10240
RT = NT // NS
E = 160000
EP = 163840
EW = EP // (NC * NS)
K = 64
NCH = EW // K


RB = 2048
EB = 8000


def _proj_body(nf_ref, hd_ref, wa_ref, ba_ref, wvs_ref, bvs_ref,
               attA_ref, attB_ref, val_ref, skip_ref):
    z = jnp.concatenate([nf_ref[...], hd_ref[...]], axis=-1)
    att = z @ wa_ref[...] + ba_ref[...]
    attA_ref[...] = att[:, :16]
    attB_ref[...] = att[:, 16:]
    vs = z @ wvs_ref[...] + bvs_ref[...]
    val_ref[...] = vs[:, :OUT]
    skip_ref[...] = vs[:, OUT:]


def _row_spec(w):
    return pl.BlockSpec((RB, w), lambda i: (i, 0))


def _full_spec(shape):
    return pl.BlockSpec(shape, lambda i: tuple(0 for _ in shape))


def _proj(nf, hd, Wa, ba, Wvs, bvs):
    return pl.pallas_call(
        _proj_body,
        grid=(NT // RB,),
        in_specs=[
            _row_spec(D), _row_spec(D),
            _full_spec((2 * D, 32)), _full_spec((1, 32)),
            _full_spec((2 * D, 2 * OUT)), _full_spec((1, 2 * OUT)),
        ],
        out_specs=[_row_spec(16), _row_spec(16), _row_spec(OUT), _row_spec(OUT)],
        out_shape=[
            jax.ShapeDtypeStruct((NT, 16), jnp.float32),
            jax.ShapeDtypeStruct((NT, 16), jnp.float32),
            jax.ShapeDtypeStruct((NT, OUT), jnp.float32),
            jax.ShapeDtypeStruct((NT, OUT), jnp.float32),
        ],
    )(nf, hd, Wa, ba, Wvs, bvs)


def _ae_body(ef_ref, w_ref, b_ref, out_ref):
    out_ref[...] = ef_ref[...] @ w_ref[...] + b_ref[...]


def _ae_proj(ef, Wae, bae):
    return pl.pallas_call(
        _ae_body,
        grid=(E // EB,),
        in_specs=[
            pl.BlockSpec((EB, 16), lambda i: (i, 0)),
            _full_spec((16, 16)), _full_spec((1, 16)),
        ],
        out_specs=pl.BlockSpec((EB, 16), lambda i: (i, 0)),
        out_shape=jax.ShapeDtypeStruct((E, 16), jnp.float32),
    )(ef, Wae, bae)


def _bridge_body(nf_ref, u_ref, den_ref, skip_ref, ex_ref,
                 wa_ref, ba_ref, wvs_ref, bvs_ref,
                 attA_ref, attB_ref, val_ref, skip2_ref):
    den128 = (den_ref[0, :, 0:8] + den_ref[1, :, 0:8]) @ ex_ref[...]
    h = jnp.maximum((u_ref[0] + u_ref[1]) / den128 + skip_ref[...], 0.0)
    z = jnp.concatenate([nf_ref[...], h], axis=-1)
    att = z @ wa_ref[...] + ba_ref[...]
    attA_ref[...] = att[:, :16]
    attB_ref[...] = att[:, 16:]
    vs = z @ wvs_ref[...] + bvs_ref[...]
    val_ref[...] = vs[:, :OUT]
    skip2_ref[...] = vs[:, OUT:]


def _pair_spec(w):
    return pl.BlockSpec((NC, RB, w), lambda i: (0, i, 0))


def _bridge(nf, u, den, skip, ex, Wa, ba, Wvs, bvs):
    return pl.pallas_call(
        _bridge_body,
        grid=(NT // RB,),
        in_specs=[
            _row_spec(D), _pair_spec(OUT), _pair_spec(16), _row_spec(OUT),
            _full_spec((H, OUT)),
            _full_spec((2 * D, 32)), _full_spec((1, 32)),
            _full_spec((2 * D, 2 * OUT)), _full_spec((1, 2 * OUT)),
        ],
        out_specs=[_row_spec(16), _row_spec(16), _row_spec(OUT), _row_spec(OUT)],
        out_shape=[
            jax.ShapeDtypeStruct((NT, 16), jnp.float32),
            jax.ShapeDtypeStruct((NT, 16), jnp.float32),
            jax.ShapeDtypeStruct((NT, OUT), jnp.float32),
            jax.ShapeDtypeStruct((NT, OUT), jnp.float32),
        ],
    )(nf, u, den, skip, ex, Wa, ba, Wvs, bvs)


def _final_body(u_ref, den_ref, skip_ref, ex_ref, out_ref):
    den128 = (den_ref[0, :, 0:8] + den_ref[1, :, 0:8]) @ ex_ref[...]
    out_ref[...] = jnp.maximum((u_ref[0] + u_ref[1]) / den128 + skip_ref[...], 0.0)


def _final(u, den, skip, ex):
    return pl.pallas_call(
        _final_body,
        grid=(NT // RB,),
        in_specs=[_pair_spec(OUT), _pair_spec(16), _row_spec(OUT),
                  _full_spec((H, OUT))],
        out_specs=_row_spec(OUT),
        out_shape=jax.ShapeDtypeStruct((NT, OUT), jnp.float32),
    )(u, den, skip, ex)


NB = 3
EP2 = EP + 2 * K


def _sc_body(swap_roles, has_ae, *refs):
    it = iter(refs)
    attA_hbm = next(it)
    attB_hbm = next(it)
    val_hbm = next(it)
    eidx_hbm = next(it)
    ae_hbm = next(it) if has_ae else None
    u_out = next(it)
    den_out = next(it)
    u_sh = next(it)
    den_sh = next(it)
    idx = [next(it) for _ in range(NB)]
    aS = [next(it) for _ in range(2)]
    aT = [next(it) for _ in range(2)]
    aeb = [next(it) for _ in range(2)] if has_ae else [None, None]
    pbuf = [next(it) for _ in range(2)]
    vrows = [next(it) for _ in range(NB)]
    sem_g = [next(it) for _ in range(NB)]
    sem_u = [next(it) for _ in range(NB)]

    c = lax.axis_index("c")
    s = lax.axis_index("s")
    wid = c * NS + s
    row0 = s * RT
    zv = jnp.zeros((16,), jnp.float32)

    def zrow(i, _):
        for j in range(8):
            vrows[0][i, pl.ds(16 * j, 16)] = zv
        pbuf[0][i, :] = zv
        return 0
    lax.fori_loop(0, K, zrow, 0)
    for b in range(RT // K):
        pltpu.sync_copy(vrows[0], u_sh.at[pl.ds(row0 + b * K, K)])
        pltpu.sync_copy(pbuf[0], den_sh.at[pl.ds(row0 + b * K, K)])

    iA, iB = (1, 0) if swap_roles else (0, 1)

    def load(j):
        b3, b2 = j % NB, j % 2
        base = wid * EW + j * K
        pltpu.sync_copy(eidx_hbm.at[:, pl.ds(base, K)], idx[b3])
        pltpu.async_copy(val_hbm.at[idx[b3].at[0]], vrows[b3], sem_g[b3])
        pltpu.async_copy(attA_hbm.at[idx[b3].at[iA]], aS[b2], sem_g[b3])
        pltpu.async_copy(attB_hbm.at[idx[b3].at[iB]], aT[b2], sem_g[b3])
        if has_ae:
            pltpu.async_copy(ae_hbm.at[pl.ds(base, K)], aeb[b2], sem_g[b3])

    def drain_gathers(j):
        b3, b2 = j % NB, j % 2
        pltpu.make_async_copy(val_hbm.at[idx[b3].at[0]], vrows[b3], sem_g[b3]).wait()
        pltpu.make_async_copy(attA_hbm.at[idx[b3].at[iA]], aS[b2], sem_g[b3]).wait()
        pltpu.make_async_copy(attB_hbm.at[idx[b3].at[iB]], aT[b2], sem_g[b3]).wait()
        if has_ae:
            pltpu.make_async_copy(ae_hbm.at[pl.ds(0, K)], aeb[b2], sem_g[b3]).wait()

    def drain_scatters(j):
        b3, b2 = j % NB, j % 2
        pltpu.make_async_copy(pbuf[b2], den_sh.at[idx[b3].at[1]], sem_u[b3]).wait()
        pltpu.make_async_copy(vrows[b3], u_sh.at[idx[b3].at[1]], sem_u[b3]).wait()

    load(0)
    load(1)
    plsc.subcore_barrier()

    for j in range(NCH):
        b3, b2 = j % NB, j % 2
        drain_gathers(j)

        @functools.partial(plsc.parallel_loop, 0, K, unroll=2)
        def _(e, _aS=aS[b2], _aT=aT[b2], _ae=aeb[b2], _p=pbuf[b2], _v=vrows[b3]):
            x = _aS[e, :] + _aT[e, :]
            if has_ae:
                x = x + _ae[e, :]
            x = jnp.maximum(x, 0.01 * x)
            pv = jnp.exp(x)
            _p[e, :] = pv
            for h in range(H):
                w = jnp.broadcast_to(pv[h], (16,))
                _v[e, pl.ds(HS * h, HS)] = _v[e, pl.ds(HS * h, HS)] * w

        pltpu.async_copy(pbuf[b2], den_sh.at[idx[b3].at[1]], sem_u[b3], add=True)
        pltpu.async_copy(vrows[b3], u_sh.at[idx[b3].at[1]], sem_u[b3], add=True)
        if j >= 1:
            drain_scatters(j - 1)
        if j + 2 < NCH:
            load(j + 2)
    drain_scatters(NCH - 1)
    plsc.subcore_barrier()

    pltpu.sync_copy(u_sh.at[pl.ds(row0, RT)], u_out.at[c, pl.ds(row0, RT)])
    pltpu.sync_copy(den_sh.at[pl.ds(row0, RT)], den_out.at[c, pl.ds(row0, RT)])


def _make_sc_phase(swap_roles, has_ae):
    scratch = [
        pltpu.VMEM_SHARED((NT, OUT), jnp.float32),
        pltpu.VMEM_SHARED((NT, 16), jnp.float32),
    ]
    scratch += [pltpu.VMEM((2, K), jnp.int32) for _ in range(NB)]
    scratch += [pltpu.VMEM((K, 16), jnp.float32) for _ in range(2)]
    scratch += [pltpu.VMEM((K, 16), jnp.float32) for _ in range(2)]
    if has_ae:
        scratch += [pltpu.VMEM((K, 16), jnp.float32) for _ in range(2)]
    scratch += [pltpu.VMEM((K, 16), jnp.float32) for _ in range(2)]
    scratch += [pltpu.VMEM((K, OUT), jnp.float32) for _ in range(NB)]
    scratch += [pltpu.SemaphoreType.DMA for _ in range(2 * NB)]
    return pl.kernel(
        functools.partial(_sc_body, swap_roles, has_ae),
        out_type=[
            jax.ShapeDtypeStruct((NC, NT, OUT), jnp.float32),
            jax.ShapeDtypeStruct((NC, NT, 16), jnp.float32),
        ],
        mesh=plsc.VectorSubcoreMesh(core_axis_name="c", subcore_axis_name="s"),
        scratch_types=scratch,
        compiler_params=pltpu.CompilerParams(use_tc_tiling_on_sc=False),
    )


_sc_phase1 = _make_sc_phase(swap_roles=False, has_ae=False)
_sc_phase2 = _make_sc_phase(swap_roles=True, has_ae=True)


def kernel(node_fts, gkt_edge_fts, hidden, cfg_indices_padded, gkt_indices_padded,
           W_m, b_m, W_skip, b_skip, W_a1, b_a1, W_a2, b_a2, W_ae, b_ae):
    nf = jnp.pad(node_fts[0], ((0, NT - N), (0, 0)))
    hd = jnp.pad(hidden[0], ((0, NT - N), (0, 0)))

    def pad_idx(a, b):
        return jnp.stack([
            jnp.pad(a, (0, EP2 - E), constant_values=N),
            jnp.pad(b, (0, EP2 - E), constant_values=N),
        ])

    cfg_eidx = pad_idx(cfg_indices_padded[0, :, 0], cfg_indices_padded[0, :, 1])
    gkt_eidx = pad_idx(gkt_indices_padded[0, :, 0], gkt_indices_padded[0, :, 1])

    zpad8 = jnp.zeros((2 * D, 8), jnp.float32)
    Wa = jnp.concatenate([W_a1, zpad8, W_a2, zpad8], axis=1)
    ba = jnp.concatenate([b_a1, jnp.zeros(8, jnp.float32),
                          b_a2, jnp.zeros(8, jnp.float32)])[None]
    Wvs = jnp.concatenate([W_m, W_skip], axis=1)
    bvs = jnp.concatenate([b_m, b_skip])[None]
    Wae = jnp.pad(W_ae, ((0, 0), (0, 8)))
    bae = jnp.pad(b_ae, (0, 8))[None]
    ex = jnp.repeat(jnp.eye(H, dtype=jnp.float32), HS, axis=1)

    ae = _ae_proj(gkt_edge_fts[0], Wae, bae)
    ae = jnp.pad(ae, ((0, EP2 - E), (0, 0)))

    attA, attB, values, skip = _proj(nf, hd, Wa, ba, Wvs, bvs)
    u, den = _sc_phase1(attA, attB, values, cfg_eidx)

    attA2, attB2, values2, skip2 = _bridge(nf, u, den, skip, ex, Wa, ba, Wvs, bvs)

    u2, den2 = _sc_phase2(attA2, attB2, values2, gkt_eidx, ae)
    ret = _final(u2, den2, skip2, ex)
    return ret[:N][None]
